# baseline (device time: 128277 ns/iter reference)
import jax
import jax.numpy as jnp
from jax import lax
from jax.experimental import pallas as pl
from jax.experimental.pallas import tpu as pltpu

N_DEV = 8


def kernel(x, w_mat, scale_x, scale_w):
    m_per, k = x.shape
    _, n_per = w_mat.shape

    def body(x_ref, w_ref, sx_ref, sw_ref, out_ref, xg_ref, send_sems, recv_sems):
        my_id = lax.axis_index("i")
        z = my_id // 4
        p = my_id % 4
        peers = [
            z * 4 + (p ^ 1),
            z * 4 + (3 - p),
            (my_id + 4) % 8,
            z * 4 + ((p + 2) % 4),
            (1 - z) * 4 + (p ^ 1),
            (1 - z) * 4 + (3 - p),
            (1 - z) * 4 + ((p + 2) % 4),
        ]

        barrier_sem = pltpu.get_barrier_semaphore()
        for t in peers:
            pl.semaphore_signal(
                barrier_sem, inc=1,
                device_id=(t,), device_id_type=pl.DeviceIdType.MESH,
            )
        pl.semaphore_wait(barrier_sem, N_DEV - 1)

        rdmas = []
        for d, t in enumerate(peers):
            rdma = pltpu.make_async_remote_copy(
                src_ref=x_ref,
                dst_ref=xg_ref.at[d],
                send_sem=send_sems.at[d],
                recv_sem=recv_sems.at[d],
                device_id=(t,),
                device_id_type=pl.DeviceIdType.MESH,
            )
            rdma.start()
            rdmas.append(rdma)

        scale = sx_ref[0] * sw_ref[0]

        def block(chunk, origin):
            acc = lax.dot_general(
                chunk, w_ref[:, :],
                dimension_numbers=(((1,), (0,)), ((), ())),
                preferred_element_type=jnp.int32,
            )
            y = acc.astype(jnp.float32) * scale
            yc = jnp.clip(y, -60.0, 60.0)
            out_ref[pl.ds(origin * m_per, m_per), :] = y / (1.0 + jnp.exp(-yc))

        block(x_ref[:, :], my_id)

        for d, t in enumerate(peers):
            rdmas[d].wait_recv()
            block(xg_ref[d], t)

        for d in range(N_DEV - 1):
            rdmas[d].wait_send()

    return pl.pallas_call(
        body,
        out_shape=jax.ShapeDtypeStruct((N_DEV * m_per, n_per), jnp.float32),
        in_specs=[
            pl.BlockSpec(memory_space=pltpu.VMEM),
            pl.BlockSpec(memory_space=pltpu.VMEM),
            pl.BlockSpec(memory_space=pltpu.SMEM),
            pl.BlockSpec(memory_space=pltpu.SMEM),
        ],
        out_specs=pl.BlockSpec(memory_space=pltpu.VMEM),
        scratch_shapes=[
            pltpu.VMEM((N_DEV - 1, m_per, k), jnp.int8),
            pltpu.SemaphoreType.DMA((N_DEV - 1,)),
            pltpu.SemaphoreType.DMA((N_DEV - 1,)),
        ],
        compiler_params=pltpu.CompilerParams(collective_id=0),
    )(x, w_mat, scale_x, scale_w)
